# baseline (speedup 1.0000x reference)
import functools

import jax
import jax.numpy as jnp
import numpy as np
from jax import lax
from jax.experimental import pallas as pl
from jax.experimental.pallas import tpu as pltpu
from jax.experimental.pallas import tpu_sc as plsc

MAXP = 524288
NLEV = 16
BRES = 16.0
MRES = 2048.0
NFEAT = 2
NPTS = 1048576

NC = 2
NS = 16
NW = NC * NS
NP = NPTS // NW
C = 128
NCH = NP // C
NCORN = 8
BIGN = MAXP

P2 = int(np.uint32(2654435761).view(np.int32))
P3 = int(np.uint32(805459861).view(np.int32))


def _levels():
    log_b = np.log(MRES / BRES) / float(NLEV - 1)
    offs, szs, scs, ress = [], [], [], []
    off = 0
    begin_fast = NLEV
    for i in range(NLEV):
        sc = BRES * np.exp(i * log_b) - 1.0
        res = int(np.uint32(np.ceil(sc))) + 1
        full = (np.ceil(sc) + 1.0) ** 3
        aligned = int((full + 7) // 8) * 8
        ps = int(min(MAXP, aligned))
        if full > ps and begin_fast == NLEV:
            begin_fast = i
        offs.append(off)
        szs.append(ps)
        scs.append(np.float32(sc))
        ress.append(res)
        off += ps
    return offs, szs, scs, ress, begin_fast, off


OFFS, SZS, SCALES, RESS, BEGIN_FAST, TOT = _levels()
SMALLN = OFFS[BEGIN_FAST]
SSEG = ((SMALLN // NS) + 7) // 8 * 8
SSEG_LAST = SMALLN - (NS - 1) * SSEG


def _body(xyz_hbm, table_hbm, scales_hbm, sconst_hbm, out_hbm,
          xc_a, xc_b, idx_a, idx_b, rows_a, rows_b, w_a, w_b, out_a, out_b,
          scales_v, sconst_v, sp_big,
          sem_xa, sem_xb, sem_ga, sem_gb, sem_oa, sem_ob):
    cid = lax.axis_index("c")
    sid = lax.axis_index("s")
    wid = sid * NC + cid
    base = wid * NP

    pltpu.sync_copy(scales_hbm, scales_v)
    pltpu.sync_copy(sconst_hbm, sconst_v)

    iota = lax.iota(jnp.int32, 16)
    zeros = jnp.zeros((16,), jnp.int32)
    ones = jnp.ones((16,), jnp.int32)

    def bcast_f(l):
        return plsc.load_gather(scales_v, [jnp.full((16,), l, jnp.int32)])

    def bcast_c(row, l):
        return plsc.load_gather(
            sconst_v, [jnp.full((16,), row * 16, jnp.int32) + l])

    def fire_xyz(k, xc_ref, sem):
        g = (base // C + k) * (3 * C)
        return pltpu.async_copy(xyz_hbm.at[pl.ds(g, 3 * C)], xc_ref, sem)

    def wait_xyz(xc_ref, sem):
        pltpu.make_async_copy(xyz_hbm.at[pl.ds(0, 3 * C)], xc_ref, sem).wait()

    def idx_pass(consts, xc_ref, idx_ref, w_ref, fast):
        if fast:
            scale, foffv = consts
        else:
            scale, resv, res2v, mv, offv = consts

        def vb(v, carry):
            s = v * 16
            x = xc_ref[pl.ds(s, 16)]
            y = xc_ref[pl.ds(s + C, 16)]
            z = xc_ref[pl.ds(s + 2 * C, 16)]
            px = x * scale + 0.5
            py = y * scale + 0.5
            pz = z * scale + 0.5
            gx = px.astype(jnp.int32)
            gy = py.astype(jnp.int32)
            gz = pz.astype(jnp.int32)
            fx = px - gx.astype(jnp.float32)
            fy = py - gy.astype(jnp.float32)
            fz = pz - gz.astype(jnp.float32)
            if fast:
                cx0 = gx
                cx1 = gx + 1
                cy0 = gy * P2
                cy1 = cy0 + P2
                cz0 = gz * P3
                cz1 = cz0 + P3
                a = [cy0 ^ cz0, cy1 ^ cz0, cy0 ^ cz1, cy1 ^ cz1]
            else:
                cx0 = gx + offv
                cx1 = cx0 + 1
                cy0 = gy * resv
                cy1 = cy0 + resv
                cz0 = gz * res2v
                cz1 = cz0 + res2v
                a = [cy0 + cz0, cy1 + cz0, cy0 + cz1, cy1 + cz1]
            wx0 = 1.0 - fx
            wy0 = 1.0 - fy
            wz0 = 1.0 - fz
            wyz = [wy0 * wz0, fy * wz0, wy0 * fz, fy * fz]
            for c in range(8):
                if fast:
                    h = (cx1 if c & 1 else cx0) ^ a[c >> 1]
                    idx = (h & (BIGN - 1)) + foffv
                else:
                    h = (cx1 if c & 1 else cx0) + a[c >> 1]
                    t = h - mv
                    idx = jnp.where(t < offv, h, t)
                idx0 = idx + idx
                idx_ref[pl.ds(c * C + s, 16)] = idx0
                idx_ref[pl.ds(NCORN * C + c * C + s, 16)] = idx0 + 1
                w = (fx if c & 1 else wx0) * wyz[c >> 1]
                w_ref[pl.ds(c * C + s, 16)] = w
            return carry

        lax.fori_loop(0, C // 16, vb, 0)

    def acc_pass(rows_ref, w_ref, out_ref):
        def vb(v, carry):
            acc0 = jnp.zeros((16,), jnp.float32)
            acc1 = jnp.zeros((16,), jnp.float32)
            for c in range(8):
                w = w_ref[pl.ds(c * C + v * 16, 16)]
                f0 = rows_ref[pl.ds(c * C + v * 16, 16)]
                f1 = rows_ref[pl.ds(NCORN * C + c * C + v * 16, 16)]
                acc0 = acc0 + w * f0
                acc1 = acc1 + w * f1
            pidx = iota + v * 16
            plsc.store_scatter(out_ref, [pidx, zeros], acc0)
            plsc.store_scatter(out_ref, [pidx, ones], acc1)
            return carry

        lax.fori_loop(0, C // 16, vb, 0)

    def fire_gather(idx_ref, rows_ref, sem):
        pltpu.async_copy(table_hbm.at[idx_ref], rows_ref, sem)

    def wait_gather(idx_ref, rows_ref, sem):
        pltpu.make_async_copy(table_hbm.at[idx_ref], rows_ref, sem).wait()

    def fire_out(l, k, out_ref, sem):
        pltpu.async_copy(out_ref, out_hbm.at[l, pl.ds(base + k * C, C)], sem)

    def wait_out(l, out_ref, sem):
        pltpu.make_async_copy(out_ref, out_hbm.at[l, pl.ds(base, C)],
                              sem).wait()

    def level_block(l, consts, fast):
        fire_xyz(0, xc_a, sem_xa)
        fire_xyz(1, xc_b, sem_xb)
        wait_xyz(xc_a, sem_xa)
        idx_pass(consts, xc_a, idx_a, w_a, fast)
        fire_gather(idx_a, rows_a, sem_ga)
        fire_xyz(2, xc_a, sem_xa)

        def body(k2, carry):
            k = 2 * k2
            wait_xyz(xc_b, sem_xb)
            idx_pass(consts, xc_b, idx_b, w_b, fast)
            fire_gather(idx_b, rows_b, sem_gb)

            @pl.when(k + 3 < NCH)
            def _():
                fire_xyz(k + 3, xc_b, sem_xb)

            wait_gather(idx_a, rows_a, sem_ga)

            @pl.when(k2 > 0)
            def _():
                wait_out(l, out_a, sem_oa)

            acc_pass(rows_a, w_a, out_a)
            fire_out(l, k, out_a, sem_oa)

            @pl.when(k + 2 < NCH)
            def _():
                wait_xyz(xc_a, sem_xa)
                idx_pass(consts, xc_a, idx_a, w_a, fast)
                fire_gather(idx_a, rows_a, sem_ga)

                @pl.when(k + 4 < NCH)
                def _():
                    fire_xyz(k + 4, xc_a, sem_xa)

            wait_gather(idx_b, rows_b, sem_gb)

            @pl.when(k2 > 0)
            def _():
                wait_out(l, out_b, sem_ob)

            acc_pass(rows_b, w_b, out_b)
            fire_out(l, k + 1, out_b, sem_ob)
            return carry

        lax.fori_loop(0, NCH // 2, body, 0)
        wait_out(l, out_a, sem_oa)
        wait_out(l, out_b, sem_ob)

    def slow_body(l, carry):
        consts = (bcast_f(l), bcast_c(0, l), bcast_c(1, l),
                  bcast_c(2, l), bcast_c(3, l))
        level_block(l, consts, False)
        return carry

    lax.fori_loop(0, BEGIN_FAST, slow_body, 0)

    def fast_body(l, carry):
        offv = jnp.full((16,), SMALLN - BEGIN_FAST * BIGN, jnp.int32) + l * BIGN
        level_block(l, (bcast_f(l), offv), True)
        return carry

    lax.fori_loop(BEGIN_FAST, NLEV, fast_body, 0)


_mesh = plsc.VectorSubcoreMesh(core_axis_name="c", subcore_axis_name="s")

_hash_kernel = functools.partial(
    pl.kernel,
    out_type=jax.ShapeDtypeStruct((NLEV, NPTS, NFEAT), jnp.float32),
    mesh=_mesh,
    compiler_params=pltpu.CompilerParams(
        needs_layout_passes=False, use_tc_tiling_on_sc=False),
    scratch_types=[
        pltpu.VMEM((3 * C,), jnp.float32),
        pltpu.VMEM((3 * C,), jnp.float32),
        pltpu.VMEM((2 * NCORN * C,), jnp.int32),
        pltpu.VMEM((2 * NCORN * C,), jnp.int32),
        pltpu.VMEM((2 * NCORN * C,), jnp.float32),
        pltpu.VMEM((2 * NCORN * C,), jnp.float32),
        pltpu.VMEM((NCORN * C,), jnp.float32),
        pltpu.VMEM((NCORN * C,), jnp.float32),
        pltpu.VMEM((C, NFEAT), jnp.float32),
        pltpu.VMEM((C, NFEAT), jnp.float32),
        pltpu.VMEM((16,), jnp.float32),
        pltpu.VMEM((4 * 16,), jnp.int32),
        pltpu.VMEM_SHARED((NFEAT * BIGN,), jnp.float32),
        pltpu.SemaphoreType.DMA,
        pltpu.SemaphoreType.DMA,
        pltpu.SemaphoreType.DMA,
        pltpu.SemaphoreType.DMA,
        pltpu.SemaphoreType.DMA,
        pltpu.SemaphoreType.DMA,
    ],
)(_body)


@jax.jit
def kernel(xyzs, table):
    xyz_c = jnp.transpose(xyzs.reshape(NPTS // C, C, 3), (0, 2, 1)).reshape(-1)
    scales = jnp.asarray(np.array(SCALES, dtype=np.float32))
    sconst = np.zeros((4, 16), dtype=np.int32)
    for l in range(BEGIN_FAST):
        sconst[0, l] = RESS[l]
        sconst[1, l] = RESS[l] * RESS[l]
        sconst[2, l] = SZS[l]
        sconst[3, l] = OFFS[l]
    sconst = jnp.asarray(sconst.reshape(-1))
    out_lbf = _hash_kernel(xyz_c, table.reshape(-1), scales, sconst)
    return jnp.transpose(out_lbf, (1, 0, 2))


# speedup vs baseline: 1.4289x; 1.4289x over previous
import functools

import jax
import jax.numpy as jnp
import numpy as np
from jax import lax
from jax.experimental import pallas as pl
from jax.experimental.pallas import tpu as pltpu
from jax.experimental.pallas import tpu_sc as plsc

MAXP = 524288
NLEV = 16
BRES = 16.0
MRES = 2048.0
NFEAT = 2
NPTS = 1048576

NC = 2
NS = 16
NW = NC * NS
NP = NPTS // NW
C = 128
NCH = NP // C
NCORN = 8
BIGN = MAXP

P2 = int(np.uint32(2654435761).view(np.int32))
P3 = int(np.uint32(805459861).view(np.int32))


def _levels():
    log_b = np.log(MRES / BRES) / float(NLEV - 1)
    offs, szs, scs, ress = [], [], [], []
    off = 0
    begin_fast = NLEV
    for i in range(NLEV):
        sc = BRES * np.exp(i * log_b) - 1.0
        res = int(np.uint32(np.ceil(sc))) + 1
        full = (np.ceil(sc) + 1.0) ** 3
        aligned = int((full + 7) // 8) * 8
        ps = int(min(MAXP, aligned))
        if full > ps and begin_fast == NLEV:
            begin_fast = i
        offs.append(off)
        szs.append(ps)
        scs.append(np.float32(sc))
        ress.append(res)
        off += ps
    return offs, szs, scs, ress, begin_fast, off


OFFS, SZS, SCALES, RESS, BEGIN_FAST, TOT = _levels()
SMALLN = OFFS[BEGIN_FAST]
SMALLW = NFEAT * SMALLN
BIGW = NFEAT * BIGN
SSEGW = ((SMALLW // NS) + 7) // 8 * 8
SSEGW_LAST = SMALLW - (NS - 1) * SSEGW
FSEGW = BIGW // NS


def _body(xyz_hbm, table_hbm, scales_hbm, sconst_hbm, out_hbm,
          xc_a, xc_b, idx_a, idx_b, rows_a, rows_b, w_a, w_b, out_a, out_b,
          scales_v, sconst_v, sp_big,
          sem_xa, sem_xb, sem_ga, sem_gb, sem_oa, sem_ob):
    cid = lax.axis_index("c")
    sid = lax.axis_index("s")
    wid = sid * NC + cid
    base = wid * NP

    pltpu.sync_copy(scales_hbm, scales_v)
    pltpu.sync_copy(sconst_hbm, sconst_v)

    @pl.when(sid < NS - 1)
    def _():
        pltpu.sync_copy(table_hbm.at[pl.ds(sid * SSEGW, SSEGW)],
                        sp_big.at[pl.ds(sid * SSEGW, SSEGW)])

    @pl.when(sid == NS - 1)
    def _():
        pltpu.sync_copy(table_hbm.at[pl.ds((NS - 1) * SSEGW, SSEGW_LAST)],
                        sp_big.at[pl.ds((NS - 1) * SSEGW, SSEGW_LAST)])

    plsc.subcore_barrier()

    iota = lax.iota(jnp.int32, 16)
    zeros = jnp.zeros((16,), jnp.int32)
    ones = jnp.ones((16,), jnp.int32)

    def bcast_f(l):
        return plsc.load_gather(scales_v, [jnp.full((16,), l, jnp.int32)])

    def bcast_c(row, l):
        return plsc.load_gather(
            sconst_v, [jnp.full((16,), row * 16, jnp.int32) + l])

    def fire_xyz(k, xc_ref, sem):
        g = (base // C + k) * (3 * C)
        return pltpu.async_copy(xyz_hbm.at[pl.ds(g, 3 * C)], xc_ref, sem)

    def wait_xyz(xc_ref, sem):
        pltpu.make_async_copy(xyz_hbm.at[pl.ds(0, 3 * C)], xc_ref, sem).wait()

    def idx_pass(consts, xc_ref, idx_ref, w_ref, fast):
        if fast:
            (scale,) = consts
        else:
            scale, resv, res2v, mv, offv = consts

        def vb(v, carry):
            s = v * 16
            x = xc_ref[pl.ds(s, 16)]
            y = xc_ref[pl.ds(s + C, 16)]
            z = xc_ref[pl.ds(s + 2 * C, 16)]
            px = x * scale + 0.5
            py = y * scale + 0.5
            pz = z * scale + 0.5
            gx = px.astype(jnp.int32)
            gy = py.astype(jnp.int32)
            gz = pz.astype(jnp.int32)
            fx = px - gx.astype(jnp.float32)
            fy = py - gy.astype(jnp.float32)
            fz = pz - gz.astype(jnp.float32)
            if fast:
                cx0 = gx
                cx1 = gx + 1
                cy0 = gy * P2
                cy1 = cy0 + P2
                cz0 = gz * P3
                cz1 = cz0 + P3
                a = [cy0 ^ cz0, cy1 ^ cz0, cy0 ^ cz1, cy1 ^ cz1]
            else:
                cx0 = gx + offv
                cx1 = cx0 + 1
                cy0 = gy * resv
                cy1 = cy0 + resv
                cz0 = gz * res2v
                cz1 = cz0 + res2v
                a = [cy0 + cz0, cy1 + cz0, cy0 + cz1, cy1 + cz1]
            wx0 = 1.0 - fx
            wy0 = 1.0 - fy
            wz0 = 1.0 - fz
            wyz = [wy0 * wz0, fy * wz0, wy0 * fz, fy * fz]
            for c in range(8):
                if fast:
                    h = (cx1 if c & 1 else cx0) ^ a[c >> 1]
                    idx = h & (BIGN - 1)
                else:
                    h = (cx1 if c & 1 else cx0) + a[c >> 1]
                    t = h - mv
                    idx = jnp.where(t < offv, h, t)
                idx0 = idx + idx
                idx_ref[pl.ds(c * C + s, 16)] = idx0
                idx_ref[pl.ds(NCORN * C + c * C + s, 16)] = idx0 + 1
                w = (fx if c & 1 else wx0) * wyz[c >> 1]
                w_ref[pl.ds(c * C + s, 16)] = w
            return carry

        lax.fori_loop(0, C // 16, vb, 0)

    def acc_pass(rows_ref, w_ref, out_ref):
        def vb(v, carry):
            acc0 = jnp.zeros((16,), jnp.float32)
            acc1 = jnp.zeros((16,), jnp.float32)
            for c in range(8):
                w = w_ref[pl.ds(c * C + v * 16, 16)]
                f0 = rows_ref[pl.ds(c * C + v * 16, 16)]
                f1 = rows_ref[pl.ds(NCORN * C + c * C + v * 16, 16)]
                acc0 = acc0 + w * f0
                acc1 = acc1 + w * f1
            pidx = iota + v * 16
            plsc.store_scatter(out_ref, [pidx, zeros], acc0)
            plsc.store_scatter(out_ref, [pidx, ones], acc1)
            return carry

        lax.fori_loop(0, C // 16, vb, 0)

    def fire_gather(idx_ref, rows_ref, sem):
        pltpu.async_copy(sp_big.at[idx_ref], rows_ref, sem)

    def wait_gather(idx_ref, rows_ref, sem):
        pltpu.make_async_copy(sp_big.at[idx_ref], rows_ref, sem).wait()

    def fire_out(l, k, out_ref, sem):
        pltpu.async_copy(out_ref, out_hbm.at[l, pl.ds(base + k * C, C)], sem)

    def wait_out(l, out_ref, sem):
        pltpu.make_async_copy(out_ref, out_hbm.at[l, pl.ds(base, C)],
                              sem).wait()

    def level_block(l, consts, fast):
        fire_xyz(0, xc_a, sem_xa)
        fire_xyz(1, xc_b, sem_xb)
        wait_xyz(xc_a, sem_xa)
        idx_pass(consts, xc_a, idx_a, w_a, fast)
        fire_gather(idx_a, rows_a, sem_ga)
        fire_xyz(2, xc_a, sem_xa)

        def body(k2, carry):
            k = 2 * k2
            wait_xyz(xc_b, sem_xb)
            idx_pass(consts, xc_b, idx_b, w_b, fast)
            fire_gather(idx_b, rows_b, sem_gb)

            @pl.when(k + 3 < NCH)
            def _():
                fire_xyz(k + 3, xc_b, sem_xb)

            wait_gather(idx_a, rows_a, sem_ga)

            @pl.when(k2 > 0)
            def _():
                wait_out(l, out_a, sem_oa)

            acc_pass(rows_a, w_a, out_a)
            fire_out(l, k, out_a, sem_oa)

            @pl.when(k + 2 < NCH)
            def _():
                wait_xyz(xc_a, sem_xa)
                idx_pass(consts, xc_a, idx_a, w_a, fast)
                fire_gather(idx_a, rows_a, sem_ga)

                @pl.when(k + 4 < NCH)
                def _():
                    fire_xyz(k + 4, xc_a, sem_xa)

            wait_gather(idx_b, rows_b, sem_gb)

            @pl.when(k2 > 0)
            def _():
                wait_out(l, out_b, sem_ob)

            acc_pass(rows_b, w_b, out_b)
            fire_out(l, k + 1, out_b, sem_ob)
            return carry

        lax.fori_loop(0, NCH // 2, body, 0)
        wait_out(l, out_a, sem_oa)
        wait_out(l, out_b, sem_ob)

    def slow_body(l, carry):
        consts = (bcast_f(l), bcast_c(0, l), bcast_c(1, l),
                  bcast_c(2, l), bcast_c(3, l))
        level_block(l, consts, False)
        return carry

    lax.fori_loop(0, BEGIN_FAST, slow_body, 0)

    def fast_body(l, carry):
        plsc.subcore_barrier()
        offw = (2 * SMALLN - 2 * BEGIN_FAST * BIGN) + l * (2 * BIGN)
        pltpu.sync_copy(table_hbm.at[pl.ds(offw + sid * FSEGW, FSEGW)],
                        sp_big.at[pl.ds(sid * FSEGW, FSEGW)])
        plsc.subcore_barrier()
        level_block(l, (bcast_f(l),), True)
        return carry

    lax.fori_loop(BEGIN_FAST, NLEV, fast_body, 0)


_mesh = plsc.VectorSubcoreMesh(core_axis_name="c", subcore_axis_name="s")

_hash_kernel = functools.partial(
    pl.kernel,
    out_type=jax.ShapeDtypeStruct((NLEV, NPTS, NFEAT), jnp.float32),
    mesh=_mesh,
    compiler_params=pltpu.CompilerParams(
        needs_layout_passes=False, use_tc_tiling_on_sc=False),
    scratch_types=[
        pltpu.VMEM((3 * C,), jnp.float32),
        pltpu.VMEM((3 * C,), jnp.float32),
        pltpu.VMEM((2 * NCORN * C,), jnp.int32),
        pltpu.VMEM((2 * NCORN * C,), jnp.int32),
        pltpu.VMEM((2 * NCORN * C,), jnp.float32),
        pltpu.VMEM((2 * NCORN * C,), jnp.float32),
        pltpu.VMEM((NCORN * C,), jnp.float32),
        pltpu.VMEM((NCORN * C,), jnp.float32),
        pltpu.VMEM((C, NFEAT), jnp.float32),
        pltpu.VMEM((C, NFEAT), jnp.float32),
        pltpu.VMEM((16,), jnp.float32),
        pltpu.VMEM((4 * 16,), jnp.int32),
        pltpu.VMEM_SHARED((NFEAT * BIGN,), jnp.float32),
        pltpu.SemaphoreType.DMA,
        pltpu.SemaphoreType.DMA,
        pltpu.SemaphoreType.DMA,
        pltpu.SemaphoreType.DMA,
        pltpu.SemaphoreType.DMA,
        pltpu.SemaphoreType.DMA,
    ],
)(_body)


@jax.jit
def kernel(xyzs, table):
    xyz_c = jnp.transpose(xyzs.reshape(NPTS // C, C, 3), (0, 2, 1)).reshape(-1)
    scales = jnp.asarray(np.array(SCALES, dtype=np.float32))
    sconst = np.zeros((4, 16), dtype=np.int32)
    for l in range(BEGIN_FAST):
        sconst[0, l] = RESS[l]
        sconst[1, l] = RESS[l] * RESS[l]
        sconst[2, l] = SZS[l]
        sconst[3, l] = OFFS[l]
    sconst = jnp.asarray(sconst.reshape(-1))
    out_lbf = _hash_kernel(xyz_c, table.reshape(-1), scales, sconst)
    return jnp.transpose(out_lbf, (1, 0, 2))


# speedup vs baseline: 1.7291x; 1.2101x over previous
import functools

import jax
import jax.numpy as jnp
import numpy as np
from jax import lax
from jax.experimental import pallas as pl
from jax.experimental.pallas import tpu as pltpu
from jax.experimental.pallas import tpu_sc as plsc

MAXP = 524288
NLEV = 16
BRES = 16.0
MRES = 2048.0
NFEAT = 2
NPTS = 1048576

NC = 2
NS = 16
NW = NC * NS
NP = NPTS // NW
C = 128
NCH = NP // C
NCORN = 8
BIGN = MAXP

P2 = int(np.uint32(2654435761).view(np.int32))
P3 = int(np.uint32(805459861).view(np.int32))


def _levels():
    log_b = np.log(MRES / BRES) / float(NLEV - 1)
    offs, szs, scs, ress = [], [], [], []
    off = 0
    begin_fast = NLEV
    for i in range(NLEV):
        sc = BRES * np.exp(i * log_b) - 1.0
        res = int(np.uint32(np.ceil(sc))) + 1
        full = (np.ceil(sc) + 1.0) ** 3
        aligned = int((full + 7) // 8) * 8
        ps = int(min(MAXP, aligned))
        if full > ps and begin_fast == NLEV:
            begin_fast = i
        offs.append(off)
        szs.append(ps)
        scs.append(np.float32(sc))
        ress.append(res)
        off += ps
    return offs, szs, scs, ress, begin_fast, off


OFFS, SZS, SCALES, RESS, BEGIN_FAST, TOT = _levels()
SMALLN = OFFS[BEGIN_FAST]
SMALLW = NFEAT * SMALLN
BIGW = NFEAT * BIGN
SSEGW = ((SMALLW // NS) + 7) // 8 * 8
SSEGW_LAST = SMALLW - (NS - 1) * SSEGW
FSEGW = BIGW // NS


def _body(xyz_hbm, table_hbm, scales_hbm, sconst_hbm, out_hbm,
          xc_a, xc_b, idx_a, idx_b, rows_a, rows_b, w_a, w_b, out_a, out_b,
          scales_v, sconst_v, sp_big,
          sem_xa, sem_xb, sem_ga, sem_gb, sem_oa, sem_ob):
    cid = lax.axis_index("c")
    sid = lax.axis_index("s")
    wid = sid * NC + cid
    base = wid * NP

    pltpu.sync_copy(scales_hbm, scales_v)
    pltpu.sync_copy(sconst_hbm, sconst_v)

    @pl.when(sid < NS - 1)
    def _():
        pltpu.sync_copy(table_hbm.at[pl.ds(sid * SSEGW, SSEGW)],
                        sp_big.at[pl.ds(sid * SSEGW, SSEGW)])

    @pl.when(sid == NS - 1)
    def _():
        pltpu.sync_copy(table_hbm.at[pl.ds((NS - 1) * SSEGW, SSEGW_LAST)],
                        sp_big.at[pl.ds((NS - 1) * SSEGW, SSEGW_LAST)])

    plsc.subcore_barrier()

    iota = lax.iota(jnp.int32, 16)
    iota3 = iota * 3
    zeros = jnp.zeros((16,), jnp.int32)
    ones = jnp.ones((16,), jnp.int32)

    def bcast_f(l):
        return plsc.load_gather(scales_v, [jnp.full((16,), l, jnp.int32)])

    def bcast_c(row, l):
        return plsc.load_gather(
            sconst_v, [jnp.full((16,), row * 16, jnp.int32) + l])

    def fire_xyz(k, xc_ref, sem):
        g = (base + k * C) * 3
        return pltpu.async_copy(xyz_hbm.at[pl.ds(g, 3 * C)], xc_ref, sem)

    def wait_xyz(xc_ref, sem):
        pltpu.make_async_copy(xyz_hbm.at[pl.ds(0, 3 * C)], xc_ref, sem).wait()

    def idx_pass(consts, xc_ref, idx_ref, w_ref, fast):
        if fast:
            (scale,) = consts
        else:
            scale, resv, res2v, mv, offv = consts

        def vb(v, carry):
            s = v * 16
            i3 = iota3 + (3 * s)
            x = plsc.load_gather(xc_ref, [i3])
            y = plsc.load_gather(xc_ref, [i3 + 1])
            z = plsc.load_gather(xc_ref, [i3 + 2])
            px = x * scale + 0.5
            py = y * scale + 0.5
            pz = z * scale + 0.5
            gx = px.astype(jnp.int32)
            gy = py.astype(jnp.int32)
            gz = pz.astype(jnp.int32)
            fx = px - gx.astype(jnp.float32)
            fy = py - gy.astype(jnp.float32)
            fz = pz - gz.astype(jnp.float32)
            if fast:
                cx0 = gx
                cx1 = gx + 1
                cy0 = gy * P2
                cy1 = cy0 + P2
                cz0 = gz * P3
                cz1 = cz0 + P3
                a = [cy0 ^ cz0, cy1 ^ cz0, cy0 ^ cz1, cy1 ^ cz1]
            else:
                cx0 = gx + offv
                cx1 = cx0 + 1
                cy0 = gy * resv
                cy1 = cy0 + resv
                cz0 = gz * res2v
                cz1 = cz0 + res2v
                a = [cy0 + cz0, cy1 + cz0, cy0 + cz1, cy1 + cz1]
            wx0 = 1.0 - fx
            wy0 = 1.0 - fy
            wz0 = 1.0 - fz
            wyz = [wy0 * wz0, fy * wz0, wy0 * fz, fy * fz]
            for c in range(8):
                if fast:
                    h = (cx1 if c & 1 else cx0) ^ a[c >> 1]
                    idx = h & (BIGN - 1)
                else:
                    h = (cx1 if c & 1 else cx0) + a[c >> 1]
                    t = h - mv
                    idx = jnp.where(t < offv, h, t)
                idx0 = idx + idx
                idx_ref[pl.ds(c * C + s, 16)] = idx0
                idx_ref[pl.ds(NCORN * C + c * C + s, 16)] = idx0 + 1
                w = (fx if c & 1 else wx0) * wyz[c >> 1]
                w_ref[pl.ds(c * C + s, 16)] = w
            return carry

        lax.fori_loop(0, C // 16, vb, 0)

    def acc_pass(rows_ref, w_ref, out_ref):
        def vb(v, carry):
            acc0 = jnp.zeros((16,), jnp.float32)
            acc1 = jnp.zeros((16,), jnp.float32)
            for c in range(8):
                w = w_ref[pl.ds(c * C + v * 16, 16)]
                f0 = rows_ref[pl.ds(c * C + v * 16, 16)]
                f1 = rows_ref[pl.ds(NCORN * C + c * C + v * 16, 16)]
                acc0 = acc0 + w * f0
                acc1 = acc1 + w * f1
            pidx = iota + v * 16
            plsc.store_scatter(out_ref, [pidx, zeros], acc0)
            plsc.store_scatter(out_ref, [pidx, ones], acc1)
            return carry

        lax.fori_loop(0, C // 16, vb, 0)

    def fire_gather(idx_ref, rows_ref, sem):
        pltpu.async_copy(sp_big.at[idx_ref], rows_ref, sem)

    def wait_gather(idx_ref, rows_ref, sem):
        pltpu.make_async_copy(sp_big.at[idx_ref], rows_ref, sem).wait()

    def fire_out(l, k, out_ref, sem):
        pltpu.async_copy(out_ref, out_hbm.at[pl.ds(base + k * C, C), l], sem)

    def wait_out(l, out_ref, sem):
        pltpu.make_async_copy(out_ref, out_hbm.at[pl.ds(base, C), l],
                              sem).wait()

    def level_block(l, consts, fast):
        fire_xyz(0, xc_a, sem_xa)
        fire_xyz(1, xc_b, sem_xb)
        wait_xyz(xc_a, sem_xa)
        idx_pass(consts, xc_a, idx_a, w_a, fast)
        fire_gather(idx_a, rows_a, sem_ga)
        fire_xyz(2, xc_a, sem_xa)

        def body(k2, carry):
            k = 2 * k2
            wait_xyz(xc_b, sem_xb)
            idx_pass(consts, xc_b, idx_b, w_b, fast)
            fire_gather(idx_b, rows_b, sem_gb)

            @pl.when(k + 3 < NCH)
            def _():
                fire_xyz(k + 3, xc_b, sem_xb)

            wait_gather(idx_a, rows_a, sem_ga)

            @pl.when(k2 > 0)
            def _():
                wait_out(l, out_a, sem_oa)

            acc_pass(rows_a, w_a, out_a)
            fire_out(l, k, out_a, sem_oa)

            @pl.when(k + 2 < NCH)
            def _():
                wait_xyz(xc_a, sem_xa)
                idx_pass(consts, xc_a, idx_a, w_a, fast)
                fire_gather(idx_a, rows_a, sem_ga)

                @pl.when(k + 4 < NCH)
                def _():
                    fire_xyz(k + 4, xc_a, sem_xa)

            wait_gather(idx_b, rows_b, sem_gb)

            @pl.when(k2 > 0)
            def _():
                wait_out(l, out_b, sem_ob)

            acc_pass(rows_b, w_b, out_b)
            fire_out(l, k + 1, out_b, sem_ob)
            return carry

        lax.fori_loop(0, NCH // 2, body, 0)
        wait_out(l, out_a, sem_oa)
        wait_out(l, out_b, sem_ob)

    def slow_body(l, carry):
        consts = (bcast_f(l), bcast_c(0, l), bcast_c(1, l),
                  bcast_c(2, l), bcast_c(3, l))
        level_block(l, consts, False)
        return carry

    lax.fori_loop(0, BEGIN_FAST, slow_body, 0)

    def fast_body(l, carry):
        plsc.subcore_barrier()
        offw = (2 * SMALLN - 2 * BEGIN_FAST * BIGN) + l * (2 * BIGN)
        pltpu.sync_copy(table_hbm.at[pl.ds(offw + sid * FSEGW, FSEGW)],
                        sp_big.at[pl.ds(sid * FSEGW, FSEGW)])
        plsc.subcore_barrier()
        level_block(l, (bcast_f(l),), True)
        return carry

    lax.fori_loop(BEGIN_FAST, NLEV, fast_body, 0)


_mesh = plsc.VectorSubcoreMesh(core_axis_name="c", subcore_axis_name="s")

_hash_kernel = functools.partial(
    pl.kernel,
    out_type=jax.ShapeDtypeStruct((NPTS, NLEV, NFEAT), jnp.float32),
    mesh=_mesh,
    compiler_params=pltpu.CompilerParams(
        needs_layout_passes=False, use_tc_tiling_on_sc=False),
    scratch_types=[
        pltpu.VMEM((3 * C,), jnp.float32),
        pltpu.VMEM((3 * C,), jnp.float32),
        pltpu.VMEM((2 * NCORN * C,), jnp.int32),
        pltpu.VMEM((2 * NCORN * C,), jnp.int32),
        pltpu.VMEM((2 * NCORN * C,), jnp.float32),
        pltpu.VMEM((2 * NCORN * C,), jnp.float32),
        pltpu.VMEM((NCORN * C,), jnp.float32),
        pltpu.VMEM((NCORN * C,), jnp.float32),
        pltpu.VMEM((C, NFEAT), jnp.float32),
        pltpu.VMEM((C, NFEAT), jnp.float32),
        pltpu.VMEM((16,), jnp.float32),
        pltpu.VMEM((4 * 16,), jnp.int32),
        pltpu.VMEM_SHARED((NFEAT * BIGN,), jnp.float32),
        pltpu.SemaphoreType.DMA,
        pltpu.SemaphoreType.DMA,
        pltpu.SemaphoreType.DMA,
        pltpu.SemaphoreType.DMA,
        pltpu.SemaphoreType.DMA,
        pltpu.SemaphoreType.DMA,
    ],
)(_body)


@jax.jit
def kernel(xyzs, table):
    xyz_c = xyzs.reshape(-1)
    scales = jnp.asarray(np.array(SCALES, dtype=np.float32))
    sconst = np.zeros((4, 16), dtype=np.int32)
    for l in range(BEGIN_FAST):
        sconst[0, l] = RESS[l]
        sconst[1, l] = RESS[l] * RESS[l]
        sconst[2, l] = SZS[l]
        sconst[3, l] = OFFS[l]
    sconst = jnp.asarray(sconst.reshape(-1))
    return _hash_kernel(xyz_c, table.reshape(-1), scales, sconst)


# speedup vs baseline: 2.8789x; 1.6650x over previous
import functools

import jax
import jax.numpy as jnp
import numpy as np
from jax import lax
from jax.experimental import pallas as pl
from jax.experimental.pallas import tpu as pltpu
from jax.experimental.pallas import tpu_sc as plsc
from jax.experimental import layout as jlayout

MAXP = 524288
NLEV = 16
BRES = 16.0
MRES = 2048.0
NFEAT = 2
NPTS = 1048576

NC = 2
NS = 16
NW = NC * NS
NP = NPTS // NW
C = 128
NCH = NP // C
NCORN = 8
BIGN = MAXP

P2 = int(np.uint32(2654435761).view(np.int32))
P3 = int(np.uint32(805459861).view(np.int32))


def _levels():
    log_b = np.log(MRES / BRES) / float(NLEV - 1)
    offs, szs, scs, ress = [], [], [], []
    off = 0
    begin_fast = NLEV
    for i in range(NLEV):
        sc = BRES * np.exp(i * log_b) - 1.0
        res = int(np.uint32(np.ceil(sc))) + 1
        full = (np.ceil(sc) + 1.0) ** 3
        aligned = int((full + 7) // 8) * 8
        ps = int(min(MAXP, aligned))
        if full > ps and begin_fast == NLEV:
            begin_fast = i
        offs.append(off)
        szs.append(ps)
        scs.append(np.float32(sc))
        ress.append(res)
        off += ps
    return offs, szs, scs, ress, begin_fast, off


OFFS, SZS, SCALES, RESS, BEGIN_FAST, TOT = _levels()
SMALLN = OFFS[BEGIN_FAST]
SMALLW = NFEAT * SMALLN
BIGW = NFEAT * BIGN
SSEGW = ((SMALLW // NS) + 7) // 8 * 8
SSEGW_LAST = SMALLW - (NS - 1) * SSEGW
FSEGW = BIGW // NS


def _body(xyz_hbm, table_hbm, scales_hbm, sconst_hbm, out_hbm,
          xc_a, xc_b, idx_a, idx_b, rows_a, rows_b, w_a, w_b, out_a, out_b,
          scales_v, sconst_v, sp_big,
          sem_xa, sem_xb, sem_ga, sem_gb, sem_oa, sem_ob):
    cid = lax.axis_index("c")
    sid = lax.axis_index("s")
    wid = sid * NC + cid
    base = wid * NP

    pltpu.sync_copy(scales_hbm, scales_v)
    pltpu.sync_copy(sconst_hbm, sconst_v)

    @pl.when(sid < NS - 1)
    def _():
        pltpu.sync_copy(table_hbm.at[pl.ds(sid * SSEGW, SSEGW)],
                        sp_big.at[pl.ds(sid * SSEGW, SSEGW)])

    @pl.when(sid == NS - 1)
    def _():
        pltpu.sync_copy(table_hbm.at[pl.ds((NS - 1) * SSEGW, SSEGW_LAST)],
                        sp_big.at[pl.ds((NS - 1) * SSEGW, SSEGW_LAST)])

    plsc.subcore_barrier()

    iota = lax.iota(jnp.int32, 16)
    iota3 = iota * 3
    zeros = jnp.zeros((16,), jnp.int32)
    ones = jnp.ones((16,), jnp.int32)

    def bcast_f(l):
        return plsc.load_gather(scales_v, [jnp.full((16,), l, jnp.int32)])

    def bcast_c(row, l):
        return plsc.load_gather(
            sconst_v, [jnp.full((16,), row * 16, jnp.int32) + l])

    def fire_xyz(k, xc_ref, sem):
        g = (base + k * C) * 3
        return pltpu.async_copy(xyz_hbm.at[pl.ds(g, 3 * C)], xc_ref, sem)

    def wait_xyz(xc_ref, sem):
        pltpu.make_async_copy(xyz_hbm.at[pl.ds(0, 3 * C)], xc_ref, sem).wait()

    def idx_pass(consts, xc_ref, idx_ref, w_ref, fast):
        if fast:
            (scale,) = consts
        else:
            scale, resv, res2v, mv, offv = consts

        def vb(v, carry):
            s = v * 16
            i3 = iota3 + (3 * s)
            x = plsc.load_gather(xc_ref, [i3])
            y = plsc.load_gather(xc_ref, [i3 + 1])
            z = plsc.load_gather(xc_ref, [i3 + 2])
            px = x * scale + 0.5
            py = y * scale + 0.5
            pz = z * scale + 0.5
            gx = px.astype(jnp.int32)
            gy = py.astype(jnp.int32)
            gz = pz.astype(jnp.int32)
            fx = px - gx.astype(jnp.float32)
            fy = py - gy.astype(jnp.float32)
            fz = pz - gz.astype(jnp.float32)
            if fast:
                cx0 = gx
                cx1 = gx + 1
                cy0 = gy * P2
                cy1 = cy0 + P2
                cz0 = gz * P3
                cz1 = cz0 + P3
                a = [cy0 ^ cz0, cy1 ^ cz0, cy0 ^ cz1, cy1 ^ cz1]
            else:
                cx0 = gx + offv
                cx1 = cx0 + 1
                cy0 = gy * resv
                cy1 = cy0 + resv
                cz0 = gz * res2v
                cz1 = cz0 + res2v
                a = [cy0 + cz0, cy1 + cz0, cy0 + cz1, cy1 + cz1]
            wx0 = 1.0 - fx
            wy0 = 1.0 - fy
            wz0 = 1.0 - fz
            wyz = [wy0 * wz0, fy * wz0, wy0 * fz, fy * fz]
            for c in range(8):
                if fast:
                    h = (cx1 if c & 1 else cx0) ^ a[c >> 1]
                    idx = h & (BIGN - 1)
                else:
                    h = (cx1 if c & 1 else cx0) + a[c >> 1]
                    t = h - mv
                    idx = jnp.where(t < offv, h, t)
                idx0 = idx + idx
                idx_ref[pl.ds(c * C + s, 16)] = idx0
                idx_ref[pl.ds(NCORN * C + c * C + s, 16)] = idx0 + 1
                w = (fx if c & 1 else wx0) * wyz[c >> 1]
                w_ref[pl.ds(c * C + s, 16)] = w
            return carry

        lax.fori_loop(0, C // 16, vb, 0)

    def acc_pass(rows_ref, w_ref, out_ref):
        def vb(v, carry):
            acc0 = jnp.zeros((16,), jnp.float32)
            acc1 = jnp.zeros((16,), jnp.float32)
            for c in range(8):
                w = w_ref[pl.ds(c * C + v * 16, 16)]
                f0 = rows_ref[pl.ds(c * C + v * 16, 16)]
                f1 = rows_ref[pl.ds(NCORN * C + c * C + v * 16, 16)]
                acc0 = acc0 + w * f0
                acc1 = acc1 + w * f1
            out_ref[pl.ds(v * 16, 16)] = acc0
            out_ref[pl.ds(C + v * 16, 16)] = acc1
            return carry

        lax.fori_loop(0, C // 16, vb, 0)

    def fire_gather(idx_ref, rows_ref, sem):
        pltpu.async_copy(sp_big.at[idx_ref], rows_ref, sem)

    def wait_gather(idx_ref, rows_ref, sem):
        pltpu.make_async_copy(sp_big.at[idx_ref], rows_ref, sem).wait()

    def fire_out(l, k, out_ref, sem):
        off = l * (NFEAT * NPTS) + (base + k * C) * NFEAT
        pltpu.async_copy(out_ref, out_hbm.at[pl.ds(off, NFEAT * C)], sem)

    def wait_out(l, out_ref, sem):
        pltpu.make_async_copy(out_ref, out_hbm.at[pl.ds(0, NFEAT * C)],
                              sem).wait()

    def level_block(l, consts, fast):
        fire_xyz(0, xc_a, sem_xa)
        fire_xyz(1, xc_b, sem_xb)
        wait_xyz(xc_a, sem_xa)
        idx_pass(consts, xc_a, idx_a, w_a, fast)
        fire_gather(idx_a, rows_a, sem_ga)
        fire_xyz(2, xc_a, sem_xa)

        def body(k2, carry):
            k = 2 * k2
            wait_xyz(xc_b, sem_xb)
            idx_pass(consts, xc_b, idx_b, w_b, fast)
            fire_gather(idx_b, rows_b, sem_gb)

            @pl.when(k + 3 < NCH)
            def _():
                fire_xyz(k + 3, xc_b, sem_xb)

            wait_gather(idx_a, rows_a, sem_ga)

            @pl.when(k2 > 0)
            def _():
                wait_out(l, out_a, sem_oa)

            acc_pass(rows_a, w_a, out_a)
            fire_out(l, k, out_a, sem_oa)

            @pl.when(k + 2 < NCH)
            def _():
                wait_xyz(xc_a, sem_xa)
                idx_pass(consts, xc_a, idx_a, w_a, fast)
                fire_gather(idx_a, rows_a, sem_ga)

                @pl.when(k + 4 < NCH)
                def _():
                    fire_xyz(k + 4, xc_a, sem_xa)

            wait_gather(idx_b, rows_b, sem_gb)

            @pl.when(k2 > 0)
            def _():
                wait_out(l, out_b, sem_ob)

            acc_pass(rows_b, w_b, out_b)
            fire_out(l, k + 1, out_b, sem_ob)
            return carry

        lax.fori_loop(0, NCH // 2, body, 0)
        wait_out(l, out_a, sem_oa)
        wait_out(l, out_b, sem_ob)

    def slow_body(l, carry):
        consts = (bcast_f(l), bcast_c(0, l), bcast_c(1, l),
                  bcast_c(2, l), bcast_c(3, l))
        level_block(l, consts, False)
        return carry

    lax.fori_loop(0, BEGIN_FAST, slow_body, 0)

    def fast_body(l, carry):
        plsc.subcore_barrier()
        offw = (2 * SMALLN - 2 * BEGIN_FAST * BIGN) + l * (2 * BIGN)
        pltpu.sync_copy(table_hbm.at[pl.ds(offw + sid * FSEGW, FSEGW)],
                        sp_big.at[pl.ds(sid * FSEGW, FSEGW)])
        plsc.subcore_barrier()
        level_block(l, (bcast_f(l),), True)
        return carry

    lax.fori_loop(BEGIN_FAST, NLEV, fast_body, 0)


_mesh = plsc.VectorSubcoreMesh(core_axis_name="c", subcore_axis_name="s")

_hash_kernel = functools.partial(
    pl.kernel,
    out_type=jax.ShapeDtypeStruct((NPTS * NLEV * NFEAT,), jnp.float32),
    mesh=_mesh,
    compiler_params=pltpu.CompilerParams(
        needs_layout_passes=False, use_tc_tiling_on_sc=False),
    scratch_types=[
        pltpu.VMEM((3 * C,), jnp.float32),
        pltpu.VMEM((3 * C,), jnp.float32),
        pltpu.VMEM((2 * NCORN * C,), jnp.int32),
        pltpu.VMEM((2 * NCORN * C,), jnp.int32),
        pltpu.VMEM((2 * NCORN * C,), jnp.float32),
        pltpu.VMEM((2 * NCORN * C,), jnp.float32),
        pltpu.VMEM((NCORN * C,), jnp.float32),
        pltpu.VMEM((NCORN * C,), jnp.float32),
        pltpu.VMEM((NFEAT * C,), jnp.float32),
        pltpu.VMEM((NFEAT * C,), jnp.float32),
        pltpu.VMEM((16,), jnp.float32),
        pltpu.VMEM((4 * 16,), jnp.int32),
        pltpu.VMEM_SHARED((NFEAT * BIGN,), jnp.float32),
        pltpu.SemaphoreType.DMA,
        pltpu.SemaphoreType.DMA,
        pltpu.SemaphoreType.DMA,
        pltpu.SemaphoreType.DMA,
        pltpu.SemaphoreType.DMA,
        pltpu.SemaphoreType.DMA,
    ],
)(_body)


@jax.jit
def kernel(xyzs, table):
    xyz_c = xyzs.reshape(-1)
    scales = jnp.asarray(np.array(SCALES, dtype=np.float32))
    sconst = np.zeros((4, 16), dtype=np.int32)
    for l in range(BEGIN_FAST):
        sconst[0, l] = RESS[l]
        sconst[1, l] = RESS[l] * RESS[l]
        sconst[2, l] = SZS[l]
        sconst[3, l] = OFFS[l]
    sconst = jnp.asarray(sconst.reshape(-1))
    out = _hash_kernel(xyz_c, table.reshape(-1), scales, sconst)
    x4 = out.reshape(NLEV, NPTS // 128, NFEAT, 128)
    return jnp.transpose(x4, (1, 3, 0, 2)).reshape(NPTS, NLEV, NFEAT)


# speedup vs baseline: 11.0395x; 3.8347x over previous
import functools

import jax
import jax.numpy as jnp
import numpy as np
from jax import lax
from jax.experimental import pallas as pl
from jax.experimental.pallas import tpu as pltpu
from jax.experimental.pallas import tpu_sc as plsc
from jax.experimental import layout as jlayout

MAXP = 524288
NLEV = 16
BRES = 16.0
MRES = 2048.0
NFEAT = 2
NPTS = 1048576

NC = 2
NS = 16
NW = NC * NS
NP = NPTS // NW
C = 128
NCH = NP // C
NCORN = 8
BIGN = MAXP

P2 = int(np.uint32(2654435761).view(np.int32))
P3 = int(np.uint32(805459861).view(np.int32))


def _levels():
    log_b = np.log(MRES / BRES) / float(NLEV - 1)
    offs, szs, scs, ress = [], [], [], []
    off = 0
    begin_fast = NLEV
    for i in range(NLEV):
        sc = BRES * np.exp(i * log_b) - 1.0
        res = int(np.uint32(np.ceil(sc))) + 1
        full = (np.ceil(sc) + 1.0) ** 3
        aligned = int((full + 7) // 8) * 8
        ps = int(min(MAXP, aligned))
        if full > ps and begin_fast == NLEV:
            begin_fast = i
        offs.append(off)
        szs.append(ps)
        scs.append(np.float32(sc))
        ress.append(res)
        off += ps
    return offs, szs, scs, ress, begin_fast, off


OFFS, SZS, SCALES, RESS, BEGIN_FAST, TOT = _levels()
SMALLN = OFFS[BEGIN_FAST]
TOTP = (TOT + 127) // 128 * 128
SMALLWP = ((SMALLN + 127) // 128) * 256
SSEGW = SMALLWP // NS
BIGWP = (BIGN // 128 + 1) * 256
FSEGW = BIGWP // NS


def _body(xyz_hbm, table_hbm, scales_hbm, sconst_hbm, out_hbm,
          xc_a, xc_b, idx_a, idx_b, rows_a, rows_b, w_a, w_b, out_a, out_b,
          scales_v, sconst_v, sp_big,
          sem_xa, sem_xb, sem_ga, sem_gb, sem_oa, sem_ob):
    cid = lax.axis_index("c")
    sid = lax.axis_index("s")
    wid = sid * NC + cid
    base = wid * NP

    pltpu.sync_copy(scales_hbm, scales_v)
    pltpu.sync_copy(sconst_hbm, sconst_v)

    pltpu.sync_copy(table_hbm.at[pl.ds(sid * SSEGW, SSEGW)],
                    sp_big.at[pl.ds(sid * SSEGW, SSEGW)])
    plsc.subcore_barrier()

    iota = lax.iota(jnp.int32, 16)
    iota3 = iota * 3
    zeros = jnp.zeros((16,), jnp.int32)
    ones = jnp.ones((16,), jnp.int32)

    def bcast_f(l):
        return plsc.load_gather(scales_v, [jnp.full((16,), l, jnp.int32)])

    def bcast_c(row, l):
        return plsc.load_gather(
            sconst_v, [jnp.full((16,), row * 16, jnp.int32) + l])

    def fire_xyz(k, xc_ref, sem):
        g = (base + k * C) * 4
        return pltpu.async_copy(xyz_hbm.at[pl.ds(g, 4 * C)], xc_ref, sem)

    def wait_xyz(xc_ref, sem):
        pltpu.make_async_copy(xyz_hbm.at[pl.ds(0, 4 * C)], xc_ref, sem).wait()

    def idx_pass(consts, xc_ref, idx_ref, w_ref, fast):
        if fast:
            scale, row0v, wsv = consts
        else:
            scale, resv, res2v, mv, offv = consts

        def vb(v, carry):
            s = v * 16
            x = xc_ref[pl.ds(s, 16)]
            y = xc_ref[pl.ds(s + C, 16)]
            z = xc_ref[pl.ds(s + 2 * C, 16)]
            px = x * scale + 0.5
            py = y * scale + 0.5
            pz = z * scale + 0.5
            gx = px.astype(jnp.int32)
            gy = py.astype(jnp.int32)
            gz = pz.astype(jnp.int32)
            fx = px - gx.astype(jnp.float32)
            fy = py - gy.astype(jnp.float32)
            fz = pz - gz.astype(jnp.float32)
            if fast:
                cx0 = gx
                cx1 = gx + 1
                cy0 = gy * P2
                cy1 = cy0 + P2
                cz0 = gz * P3
                cz1 = cz0 + P3
                a = [cy0 ^ cz0, cy1 ^ cz0, cy0 ^ cz1, cy1 ^ cz1]
            else:
                cx0 = gx + offv
                cx1 = cx0 + 1
                cy0 = gy * resv
                cy1 = cy0 + resv
                cz0 = gz * res2v
                cz1 = cz0 + res2v
                a = [cy0 + cz0, cy1 + cz0, cy0 + cz1, cy1 + cz1]
            wx0 = 1.0 - fx
            wy0 = 1.0 - fy
            wz0 = 1.0 - fz
            wyz = [wy0 * wz0, fy * wz0, wy0 * fz, fy * fz]
            for c in range(8):
                if fast:
                    h = (cx1 if c & 1 else cx0) ^ a[c >> 1]
                    idx = (h & (BIGN - 1)) + row0v
                else:
                    h = (cx1 if c & 1 else cx0) + a[c >> 1]
                    t = h - mv
                    idx = jnp.where(t < offv, h, t)
                idx0 = ((idx >> 7) << 8) + (idx & 127)
                if fast:
                    idx0 = idx0 - wsv
                idx_ref[pl.ds(c * C + s, 16)] = idx0
                idx_ref[pl.ds(NCORN * C + c * C + s, 16)] = idx0 + 128
                w = (fx if c & 1 else wx0) * wyz[c >> 1]
                w_ref[pl.ds(c * C + s, 16)] = w
            return carry

        lax.fori_loop(0, C // 16, vb, 0)

    def acc_pass(rows_ref, w_ref, out_ref):
        def vb(v, carry):
            acc0 = jnp.zeros((16,), jnp.float32)
            acc1 = jnp.zeros((16,), jnp.float32)
            for c in range(8):
                w = w_ref[pl.ds(c * C + v * 16, 16)]
                f0 = rows_ref[pl.ds(c * C + v * 16, 16)]
                f1 = rows_ref[pl.ds(NCORN * C + c * C + v * 16, 16)]
                acc0 = acc0 + w * f0
                acc1 = acc1 + w * f1
            out_ref[pl.ds(v * 16, 16)] = acc0
            out_ref[pl.ds(C + v * 16, 16)] = acc1
            return carry

        lax.fori_loop(0, C // 16, vb, 0)

    def fire_gather(idx_ref, rows_ref, sem):
        pltpu.async_copy(sp_big.at[idx_ref], rows_ref, sem)

    def wait_gather(idx_ref, rows_ref, sem):
        pltpu.make_async_copy(sp_big.at[idx_ref], rows_ref, sem).wait()

    def fire_out(l, k, out_ref, sem):
        off = l * (NFEAT * NPTS) + (base + k * C) * NFEAT
        pltpu.async_copy(out_ref, out_hbm.at[pl.ds(off, NFEAT * C)], sem)

    def wait_out(l, out_ref, sem):
        pltpu.make_async_copy(out_ref, out_hbm.at[pl.ds(0, NFEAT * C)],
                              sem).wait()

    def level_block(l, consts, fast):
        fire_xyz(0, xc_a, sem_xa)
        fire_xyz(1, xc_b, sem_xb)
        wait_xyz(xc_a, sem_xa)
        idx_pass(consts, xc_a, idx_a, w_a, fast)
        fire_gather(idx_a, rows_a, sem_ga)
        fire_xyz(2, xc_a, sem_xa)

        def body(k2, carry):
            k = 2 * k2
            wait_xyz(xc_b, sem_xb)
            idx_pass(consts, xc_b, idx_b, w_b, fast)
            fire_gather(idx_b, rows_b, sem_gb)

            @pl.when(k + 3 < NCH)
            def _():
                fire_xyz(k + 3, xc_b, sem_xb)

            wait_gather(idx_a, rows_a, sem_ga)

            @pl.when(k2 > 0)
            def _():
                wait_out(l, out_a, sem_oa)

            acc_pass(rows_a, w_a, out_a)
            fire_out(l, k, out_a, sem_oa)

            @pl.when(k + 2 < NCH)
            def _():
                wait_xyz(xc_a, sem_xa)
                idx_pass(consts, xc_a, idx_a, w_a, fast)
                fire_gather(idx_a, rows_a, sem_ga)

                @pl.when(k + 4 < NCH)
                def _():
                    fire_xyz(k + 4, xc_a, sem_xa)

            wait_gather(idx_b, rows_b, sem_gb)

            @pl.when(k2 > 0)
            def _():
                wait_out(l, out_b, sem_ob)

            acc_pass(rows_b, w_b, out_b)
            fire_out(l, k + 1, out_b, sem_ob)
            return carry

        lax.fori_loop(0, NCH // 2, body, 0)
        wait_out(l, out_a, sem_oa)
        wait_out(l, out_b, sem_ob)

    def slow_body(l, carry):
        consts = (bcast_f(l), bcast_c(0, l), bcast_c(1, l),
                  bcast_c(2, l), bcast_c(3, l))
        level_block(l, consts, False)
        return carry

    lax.fori_loop(0, BEGIN_FAST, slow_body, 0)

    def fast_body(l, carry):
        plsc.subcore_barrier()
        row0 = (SMALLN - BEGIN_FAST * BIGN) + l * BIGN
        ws = pl.multiple_of((row0 - 72) * 2, 256)
        pltpu.sync_copy(table_hbm.at[pl.ds(ws + sid * FSEGW, FSEGW)],
                        sp_big.at[pl.ds(sid * FSEGW, FSEGW)])
        plsc.subcore_barrier()
        row0v = jnp.zeros((16,), jnp.int32) + row0
        wsv = jnp.zeros((16,), jnp.int32) + ws
        level_block(l, (bcast_f(l), row0v, wsv), True)
        return carry

    lax.fori_loop(BEGIN_FAST, NLEV, fast_body, 0)


_mesh = plsc.VectorSubcoreMesh(core_axis_name="c", subcore_axis_name="s")

_hash_kernel = functools.partial(
    pl.kernel,
    out_type=jax.ShapeDtypeStruct((NPTS * NLEV * NFEAT,), jnp.float32),
    mesh=_mesh,
    compiler_params=pltpu.CompilerParams(
        needs_layout_passes=False, use_tc_tiling_on_sc=False),
    scratch_types=[
        pltpu.VMEM((4 * C,), jnp.float32),
        pltpu.VMEM((4 * C,), jnp.float32),
        pltpu.VMEM((2 * NCORN * C,), jnp.int32),
        pltpu.VMEM((2 * NCORN * C,), jnp.int32),
        pltpu.VMEM((2 * NCORN * C,), jnp.float32),
        pltpu.VMEM((2 * NCORN * C,), jnp.float32),
        pltpu.VMEM((NCORN * C,), jnp.float32),
        pltpu.VMEM((NCORN * C,), jnp.float32),
        pltpu.VMEM((NFEAT * C,), jnp.float32),
        pltpu.VMEM((NFEAT * C,), jnp.float32),
        pltpu.VMEM((16,), jnp.float32),
        pltpu.VMEM((4 * 16,), jnp.int32),
        pltpu.VMEM_SHARED((BIGWP,), jnp.float32),
        pltpu.SemaphoreType.DMA,
        pltpu.SemaphoreType.DMA,
        pltpu.SemaphoreType.DMA,
        pltpu.SemaphoreType.DMA,
        pltpu.SemaphoreType.DMA,
        pltpu.SemaphoreType.DMA,
    ],
)(_body)


@jax.jit
def kernel(xyzs, table):
    xyz_p = jnp.pad(xyzs.reshape(NPTS // 128, 128, 3), ((0, 0), (0, 0), (0, 1)))
    xyz_c = jnp.transpose(xyz_p, (0, 2, 1)).reshape(-1)
    tab_p = jnp.pad(table, ((0, TOTP - TOT), (0, 0)))
    tab_c = jnp.transpose(tab_p.reshape(TOTP // 128, 128, 2), (0, 2, 1))
    tab_c = tab_c.reshape(-1)
    scales = jnp.asarray(np.array(SCALES, dtype=np.float32))
    sconst = np.zeros((4, 16), dtype=np.int32)
    for l in range(BEGIN_FAST):
        sconst[0, l] = RESS[l]
        sconst[1, l] = RESS[l] * RESS[l]
        sconst[2, l] = SZS[l]
        sconst[3, l] = OFFS[l]
    sconst = jnp.asarray(sconst.reshape(-1))
    out = _hash_kernel(xyz_c, tab_c, scales, sconst)
    x4 = out.reshape(NLEV, NPTS // 128, NFEAT, 128)
    return jnp.transpose(x4, (1, 3, 0, 2)).reshape(NPTS, NLEV, NFEAT)


# speedup vs baseline: 11.4263x; 1.0350x over previous
import functools

import jax
import jax.numpy as jnp
import numpy as np
from jax import lax
from jax.experimental import pallas as pl
from jax.experimental.pallas import tpu as pltpu
from jax.experimental.pallas import tpu_sc as plsc
from jax.experimental import layout as jlayout

MAXP = 524288
NLEV = 16
BRES = 16.0
MRES = 2048.0
NFEAT = 2
NPTS = 1048576

NC = 2
NS = 16
NW = NC * NS
NP = NPTS // NW
C = 256
NCH = NP // C
NCORN = 8
BIGN = MAXP

P2 = int(np.uint32(2654435761).view(np.int32))
P3 = int(np.uint32(805459861).view(np.int32))


def _levels():
    log_b = np.log(MRES / BRES) / float(NLEV - 1)
    offs, szs, scs, ress = [], [], [], []
    off = 0
    begin_fast = NLEV
    for i in range(NLEV):
        sc = BRES * np.exp(i * log_b) - 1.0
        res = int(np.uint32(np.ceil(sc))) + 1
        full = (np.ceil(sc) + 1.0) ** 3
        aligned = int((full + 7) // 8) * 8
        ps = int(min(MAXP, aligned))
        if full > ps and begin_fast == NLEV:
            begin_fast = i
        offs.append(off)
        szs.append(ps)
        scs.append(np.float32(sc))
        ress.append(res)
        off += ps
    return offs, szs, scs, ress, begin_fast, off


OFFS, SZS, SCALES, RESS, BEGIN_FAST, TOT = _levels()
SMALLN = OFFS[BEGIN_FAST]
TOTP = (TOT + 127) // 128 * 128
SMALLWP = ((SMALLN + 127) // 128) * 256
SSEGW = SMALLWP // NS
BIGWP = (BIGN // 128 + 1) * 256
FSEGW = BIGWP // NS


def _body(xyz_hbm, table_hbm, scales_hbm, sconst_hbm, out_hbm,
          xc_a, xc_b, idx_a, idx_b, rows_a, rows_b, w_a, w_b, out_a, out_b,
          scales_v, sconst_v, sp_big,
          sem_xa, sem_xb, sem_ga, sem_gb, sem_oa, sem_ob):
    cid = lax.axis_index("c")
    sid = lax.axis_index("s")
    wid = sid * NC + cid
    base = wid * NP

    pltpu.sync_copy(scales_hbm, scales_v)
    pltpu.sync_copy(sconst_hbm, sconst_v)

    pltpu.sync_copy(table_hbm.at[pl.ds(sid * SSEGW, SSEGW)],
                    sp_big.at[pl.ds(sid * SSEGW, SSEGW)])
    plsc.subcore_barrier()

    iota = lax.iota(jnp.int32, 16)
    iota3 = iota * 3
    zeros = jnp.zeros((16,), jnp.int32)
    ones = jnp.ones((16,), jnp.int32)

    def bcast_f(l):
        return plsc.load_gather(scales_v, [jnp.full((16,), l, jnp.int32)])

    def bcast_c(row, l):
        return plsc.load_gather(
            sconst_v, [jnp.full((16,), row * 16, jnp.int32) + l])

    def fire_xyz(k, xc_ref, sem):
        g = (base + k * C) * 4
        return pltpu.async_copy(xyz_hbm.at[pl.ds(g, 4 * C)], xc_ref, sem)

    def wait_xyz(xc_ref, sem):
        pltpu.make_async_copy(xyz_hbm.at[pl.ds(0, 4 * C)], xc_ref, sem).wait()

    def idx_pass(consts, xc_ref, idx_ref, w_ref, fast):
        if fast:
            scale, row0v, wsv = consts
        else:
            scale, resv, res2v, mv, offv = consts

        def vb(v, carry):
            s = v * 16
            sb = (v >> 3) * 512 + (v & 7) * 16
            x = xc_ref[pl.ds(sb, 16)]
            y = xc_ref[pl.ds(sb + 128, 16)]
            z = xc_ref[pl.ds(sb + 256, 16)]
            px = x * scale + 0.5
            py = y * scale + 0.5
            pz = z * scale + 0.5
            gx = px.astype(jnp.int32)
            gy = py.astype(jnp.int32)
            gz = pz.astype(jnp.int32)
            fx = px - gx.astype(jnp.float32)
            fy = py - gy.astype(jnp.float32)
            fz = pz - gz.astype(jnp.float32)
            if fast:
                cx0 = gx
                cx1 = gx + 1
                cy0 = gy * P2
                cy1 = cy0 + P2
                cz0 = gz * P3
                cz1 = cz0 + P3
                a = [cy0 ^ cz0, cy1 ^ cz0, cy0 ^ cz1, cy1 ^ cz1]
            else:
                cx0 = gx + offv
                cx1 = cx0 + 1
                cy0 = gy * resv
                cy1 = cy0 + resv
                cz0 = gz * res2v
                cz1 = cz0 + res2v
                a = [cy0 + cz0, cy1 + cz0, cy0 + cz1, cy1 + cz1]
            wx0 = 1.0 - fx
            wy0 = 1.0 - fy
            wz0 = 1.0 - fz
            wyz = [wy0 * wz0, fy * wz0, wy0 * fz, fy * fz]
            for c in range(8):
                if fast:
                    h = (cx1 if c & 1 else cx0) ^ a[c >> 1]
                    idx = (h & (BIGN - 1)) + row0v
                else:
                    h = (cx1 if c & 1 else cx0) + a[c >> 1]
                    t = h - mv
                    idx = jnp.where(t < offv, h, t)
                idx0 = ((idx >> 7) << 8) + (idx & 127)
                if fast:
                    idx0 = idx0 - wsv
                idx_ref[pl.ds(c * C + s, 16)] = idx0
                idx_ref[pl.ds(NCORN * C + c * C + s, 16)] = idx0 + 128
                w = (fx if c & 1 else wx0) * wyz[c >> 1]
                w_ref[pl.ds(c * C + s, 16)] = w
            return carry

        lax.fori_loop(0, C // 16, vb, 0)

    def acc_pass(rows_ref, w_ref, out_ref):
        def vb(v, carry):
            acc0 = jnp.zeros((16,), jnp.float32)
            acc1 = jnp.zeros((16,), jnp.float32)
            for c in range(8):
                w = w_ref[pl.ds(c * C + v * 16, 16)]
                f0 = rows_ref[pl.ds(c * C + v * 16, 16)]
                f1 = rows_ref[pl.ds(NCORN * C + c * C + v * 16, 16)]
                acc0 = acc0 + w * f0
                acc1 = acc1 + w * f1
            ob = (v >> 3) * 256 + (v & 7) * 16
            out_ref[pl.ds(ob, 16)] = acc0
            out_ref[pl.ds(ob + 128, 16)] = acc1
            return carry

        lax.fori_loop(0, C // 16, vb, 0)

    def fire_gather(idx_ref, rows_ref, sem):
        pltpu.async_copy(sp_big.at[idx_ref], rows_ref, sem)

    def wait_gather(idx_ref, rows_ref, sem):
        pltpu.make_async_copy(sp_big.at[idx_ref], rows_ref, sem).wait()

    def fire_out(l, k, out_ref, sem):
        off = l * (NFEAT * NPTS) + (base + k * C) * NFEAT
        pltpu.async_copy(out_ref, out_hbm.at[pl.ds(off, NFEAT * C)], sem)

    def wait_out(l, out_ref, sem):
        pltpu.make_async_copy(out_ref, out_hbm.at[pl.ds(0, NFEAT * C)],
                              sem).wait()

    def level_block(l, consts, fast):
        fire_xyz(0, xc_a, sem_xa)
        fire_xyz(1, xc_b, sem_xb)
        wait_xyz(xc_a, sem_xa)
        idx_pass(consts, xc_a, idx_a, w_a, fast)
        fire_gather(idx_a, rows_a, sem_ga)
        fire_xyz(2, xc_a, sem_xa)

        def body(k2, carry):
            k = 2 * k2
            wait_xyz(xc_b, sem_xb)
            idx_pass(consts, xc_b, idx_b, w_b, fast)
            fire_gather(idx_b, rows_b, sem_gb)

            @pl.when(k + 3 < NCH)
            def _():
                fire_xyz(k + 3, xc_b, sem_xb)

            wait_gather(idx_a, rows_a, sem_ga)

            @pl.when(k2 > 0)
            def _():
                wait_out(l, out_a, sem_oa)

            acc_pass(rows_a, w_a, out_a)
            fire_out(l, k, out_a, sem_oa)

            @pl.when(k + 2 < NCH)
            def _():
                wait_xyz(xc_a, sem_xa)
                idx_pass(consts, xc_a, idx_a, w_a, fast)
                fire_gather(idx_a, rows_a, sem_ga)

                @pl.when(k + 4 < NCH)
                def _():
                    fire_xyz(k + 4, xc_a, sem_xa)

            wait_gather(idx_b, rows_b, sem_gb)

            @pl.when(k2 > 0)
            def _():
                wait_out(l, out_b, sem_ob)

            acc_pass(rows_b, w_b, out_b)
            fire_out(l, k + 1, out_b, sem_ob)
            return carry

        lax.fori_loop(0, NCH // 2, body, 0)
        wait_out(l, out_a, sem_oa)
        wait_out(l, out_b, sem_ob)

    def slow_body(l, carry):
        consts = (bcast_f(l), bcast_c(0, l), bcast_c(1, l),
                  bcast_c(2, l), bcast_c(3, l))
        level_block(l, consts, False)
        return carry

    lax.fori_loop(0, BEGIN_FAST, slow_body, 0)

    def fast_body(l, carry):
        plsc.subcore_barrier()
        row0 = (SMALLN - BEGIN_FAST * BIGN) + l * BIGN
        ws = pl.multiple_of((row0 - 72) * 2, 256)
        pltpu.sync_copy(table_hbm.at[pl.ds(ws + sid * FSEGW, FSEGW)],
                        sp_big.at[pl.ds(sid * FSEGW, FSEGW)])
        plsc.subcore_barrier()
        row0v = jnp.zeros((16,), jnp.int32) + row0
        wsv = jnp.zeros((16,), jnp.int32) + ws
        level_block(l, (bcast_f(l), row0v, wsv), True)
        return carry

    lax.fori_loop(BEGIN_FAST, NLEV, fast_body, 0)


_mesh = plsc.VectorSubcoreMesh(core_axis_name="c", subcore_axis_name="s")

_hash_kernel = functools.partial(
    pl.kernel,
    out_type=jax.ShapeDtypeStruct((NPTS * NLEV * NFEAT,), jnp.float32),
    mesh=_mesh,
    compiler_params=pltpu.CompilerParams(
        needs_layout_passes=False, use_tc_tiling_on_sc=False),
    scratch_types=[
        pltpu.VMEM((4 * C,), jnp.float32),
        pltpu.VMEM((4 * C,), jnp.float32),
        pltpu.VMEM((2 * NCORN * C,), jnp.int32),
        pltpu.VMEM((2 * NCORN * C,), jnp.int32),
        pltpu.VMEM((2 * NCORN * C,), jnp.float32),
        pltpu.VMEM((2 * NCORN * C,), jnp.float32),
        pltpu.VMEM((NCORN * C,), jnp.float32),
        pltpu.VMEM((NCORN * C,), jnp.float32),
        pltpu.VMEM((NFEAT * C,), jnp.float32),
        pltpu.VMEM((NFEAT * C,), jnp.float32),
        pltpu.VMEM((16,), jnp.float32),
        pltpu.VMEM((4 * 16,), jnp.int32),
        pltpu.VMEM_SHARED((BIGWP,), jnp.float32),
        pltpu.SemaphoreType.DMA,
        pltpu.SemaphoreType.DMA,
        pltpu.SemaphoreType.DMA,
        pltpu.SemaphoreType.DMA,
        pltpu.SemaphoreType.DMA,
        pltpu.SemaphoreType.DMA,
    ],
)(_body)


@jax.jit
def kernel(xyzs, table):
    xyz_p = jnp.pad(xyzs.reshape(NPTS // 128, 128, 3), ((0, 0), (0, 0), (0, 1)))
    xyz_c = jnp.transpose(xyz_p, (0, 2, 1)).reshape(-1)
    tab_p = jnp.pad(table, ((0, TOTP - TOT), (0, 0)))
    tab_c = jnp.transpose(tab_p.reshape(TOTP // 128, 128, 2), (0, 2, 1))
    tab_c = tab_c.reshape(-1)
    scales = jnp.asarray(np.array(SCALES, dtype=np.float32))
    sconst = np.zeros((4, 16), dtype=np.int32)
    for l in range(BEGIN_FAST):
        sconst[0, l] = RESS[l]
        sconst[1, l] = RESS[l] * RESS[l]
        sconst[2, l] = SZS[l]
        sconst[3, l] = OFFS[l]
    sconst = jnp.asarray(sconst.reshape(-1))
    out = _hash_kernel(xyz_c, tab_c, scales, sconst)
    x4 = out.reshape(NLEV, NPTS // 128, NFEAT, 128)
    return jnp.transpose(x4, (1, 3, 0, 2)).reshape(NPTS, NLEV, NFEAT)


# speedup vs baseline: 11.5844x; 1.0138x over previous
import functools

import jax
import jax.numpy as jnp
import numpy as np
from jax import lax
from jax.experimental import pallas as pl
from jax.experimental.pallas import tpu as pltpu
from jax.experimental.pallas import tpu_sc as plsc
from jax.experimental import layout as jlayout

MAXP = 524288
NLEV = 16
BRES = 16.0
MRES = 2048.0
NFEAT = 2
NPTS = 1048576

NC = 2
NS = 16
NW = NC * NS
NP = NPTS // NW
C = 512
NCH = NP // C
NCORN = 8
BIGN = MAXP

P2 = int(np.uint32(2654435761).view(np.int32))
P3 = int(np.uint32(805459861).view(np.int32))


def _levels():
    log_b = np.log(MRES / BRES) / float(NLEV - 1)
    offs, szs, scs, ress = [], [], [], []
    off = 0
    begin_fast = NLEV
    for i in range(NLEV):
        sc = BRES * np.exp(i * log_b) - 1.0
        res = int(np.uint32(np.ceil(sc))) + 1
        full = (np.ceil(sc) + 1.0) ** 3
        aligned = int((full + 7) // 8) * 8
        ps = int(min(MAXP, aligned))
        if full > ps and begin_fast == NLEV:
            begin_fast = i
        offs.append(off)
        szs.append(ps)
        scs.append(np.float32(sc))
        ress.append(res)
        off += ps
    return offs, szs, scs, ress, begin_fast, off


OFFS, SZS, SCALES, RESS, BEGIN_FAST, TOT = _levels()
SMALLN = OFFS[BEGIN_FAST]
TOTP = (TOT + 127) // 128 * 128
SMALLWP = ((SMALLN + 127) // 128) * 256
SSEGW = SMALLWP // NS
BIGWP = (BIGN // 128 + 1) * 256
FSEGW = BIGWP // NS


def _body(xyz_hbm, table_hbm, scales_hbm, sconst_hbm, out_hbm,
          xc_a, xc_b, idx_a, idx_b, rows_a, rows_b, w_a, w_b, out_a, out_b,
          scales_v, sconst_v, sp_big,
          sem_xa, sem_xb, sem_ga, sem_gb, sem_oa, sem_ob):
    cid = lax.axis_index("c")
    sid = lax.axis_index("s")
    wid = sid * NC + cid
    base = wid * NP

    pltpu.sync_copy(scales_hbm, scales_v)
    pltpu.sync_copy(sconst_hbm, sconst_v)

    pltpu.sync_copy(table_hbm.at[pl.ds(sid * SSEGW, SSEGW)],
                    sp_big.at[pl.ds(sid * SSEGW, SSEGW)])
    plsc.subcore_barrier()

    iota = lax.iota(jnp.int32, 16)
    iota3 = iota * 3
    zeros = jnp.zeros((16,), jnp.int32)
    ones = jnp.ones((16,), jnp.int32)

    def bcast_f(l):
        return plsc.load_gather(scales_v, [jnp.full((16,), l, jnp.int32)])

    def bcast_c(row, l):
        return plsc.load_gather(
            sconst_v, [jnp.full((16,), row * 16, jnp.int32) + l])

    def fire_xyz(k, xc_ref, sem):
        g = (base + k * C) * 4
        return pltpu.async_copy(xyz_hbm.at[pl.ds(g, 4 * C)], xc_ref, sem)

    def wait_xyz(xc_ref, sem):
        pltpu.make_async_copy(xyz_hbm.at[pl.ds(0, 4 * C)], xc_ref, sem).wait()

    def idx_pass(consts, xc_ref, idx_ref, w_ref, fast):
        if fast:
            scale, row0v, wsv = consts
        else:
            scale, resv, res2v, mv, offv = consts

        def vb(v, carry):
            s = v * 16
            sb = (v >> 3) * 512 + (v & 7) * 16
            x = xc_ref[pl.ds(sb, 16)]
            y = xc_ref[pl.ds(sb + 128, 16)]
            z = xc_ref[pl.ds(sb + 256, 16)]
            px = x * scale + 0.5
            py = y * scale + 0.5
            pz = z * scale + 0.5
            gx = px.astype(jnp.int32)
            gy = py.astype(jnp.int32)
            gz = pz.astype(jnp.int32)
            fx = px - gx.astype(jnp.float32)
            fy = py - gy.astype(jnp.float32)
            fz = pz - gz.astype(jnp.float32)
            if fast:
                cx0 = gx
                cx1 = gx + 1
                cy0 = gy * P2
                cy1 = cy0 + P2
                cz0 = gz * P3
                cz1 = cz0 + P3
                a = [cy0 ^ cz0, cy1 ^ cz0, cy0 ^ cz1, cy1 ^ cz1]
            else:
                cx0 = gx + offv
                cx1 = cx0 + 1
                cy0 = gy * resv
                cy1 = cy0 + resv
                cz0 = gz * res2v
                cz1 = cz0 + res2v
                a = [cy0 + cz0, cy1 + cz0, cy0 + cz1, cy1 + cz1]
            wx0 = 1.0 - fx
            wy0 = 1.0 - fy
            wz0 = 1.0 - fz
            wyz = [wy0 * wz0, fy * wz0, wy0 * fz, fy * fz]
            for c in range(8):
                if fast:
                    h = (cx1 if c & 1 else cx0) ^ a[c >> 1]
                    idx = (h & (BIGN - 1)) + row0v
                else:
                    h = (cx1 if c & 1 else cx0) + a[c >> 1]
                    t = h - mv
                    idx = jnp.where(t < offv, h, t)
                idx0 = ((idx >> 7) << 8) + (idx & 127)
                if fast:
                    idx0 = idx0 - wsv
                idx_ref[pl.ds(c * C + s, 16)] = idx0
                idx_ref[pl.ds(NCORN * C + c * C + s, 16)] = idx0 + 128
                w = (fx if c & 1 else wx0) * wyz[c >> 1]
                w_ref[pl.ds(c * C + s, 16)] = w
            return carry

        lax.fori_loop(0, C // 16, vb, 0)

    def acc_pass(rows_ref, w_ref, out_ref):
        def vb(v, carry):
            acc0 = jnp.zeros((16,), jnp.float32)
            acc1 = jnp.zeros((16,), jnp.float32)
            for c in range(8):
                w = w_ref[pl.ds(c * C + v * 16, 16)]
                f0 = rows_ref[pl.ds(c * C + v * 16, 16)]
                f1 = rows_ref[pl.ds(NCORN * C + c * C + v * 16, 16)]
                acc0 = acc0 + w * f0
                acc1 = acc1 + w * f1
            ob = (v >> 3) * 256 + (v & 7) * 16
            out_ref[pl.ds(ob, 16)] = acc0
            out_ref[pl.ds(ob + 128, 16)] = acc1
            return carry

        lax.fori_loop(0, C // 16, vb, 0)

    def fire_gather(idx_ref, rows_ref, sem):
        pltpu.async_copy(sp_big.at[idx_ref], rows_ref, sem)

    def wait_gather(idx_ref, rows_ref, sem):
        pltpu.make_async_copy(sp_big.at[idx_ref], rows_ref, sem).wait()

    def fire_out(l, k, out_ref, sem):
        off = l * (NFEAT * NPTS) + (base + k * C) * NFEAT
        pltpu.async_copy(out_ref, out_hbm.at[pl.ds(off, NFEAT * C)], sem)

    def wait_out(l, out_ref, sem):
        pltpu.make_async_copy(out_ref, out_hbm.at[pl.ds(0, NFEAT * C)],
                              sem).wait()

    def level_block(l, consts, fast):
        fire_xyz(0, xc_a, sem_xa)
        fire_xyz(1, xc_b, sem_xb)
        wait_xyz(xc_a, sem_xa)
        idx_pass(consts, xc_a, idx_a, w_a, fast)
        fire_gather(idx_a, rows_a, sem_ga)
        fire_xyz(2, xc_a, sem_xa)

        def body(k2, carry):
            k = 2 * k2
            wait_xyz(xc_b, sem_xb)
            idx_pass(consts, xc_b, idx_b, w_b, fast)
            fire_gather(idx_b, rows_b, sem_gb)

            @pl.when(k + 3 < NCH)
            def _():
                fire_xyz(k + 3, xc_b, sem_xb)

            wait_gather(idx_a, rows_a, sem_ga)

            @pl.when(k2 > 0)
            def _():
                wait_out(l, out_a, sem_oa)

            acc_pass(rows_a, w_a, out_a)
            fire_out(l, k, out_a, sem_oa)

            @pl.when(k + 2 < NCH)
            def _():
                wait_xyz(xc_a, sem_xa)
                idx_pass(consts, xc_a, idx_a, w_a, fast)
                fire_gather(idx_a, rows_a, sem_ga)

                @pl.when(k + 4 < NCH)
                def _():
                    fire_xyz(k + 4, xc_a, sem_xa)

            wait_gather(idx_b, rows_b, sem_gb)

            @pl.when(k2 > 0)
            def _():
                wait_out(l, out_b, sem_ob)

            acc_pass(rows_b, w_b, out_b)
            fire_out(l, k + 1, out_b, sem_ob)
            return carry

        lax.fori_loop(0, NCH // 2, body, 0)
        wait_out(l, out_a, sem_oa)
        wait_out(l, out_b, sem_ob)

    def slow_body(l, carry):
        consts = (bcast_f(l), bcast_c(0, l), bcast_c(1, l),
                  bcast_c(2, l), bcast_c(3, l))
        level_block(l, consts, False)
        return carry

    lax.fori_loop(0, BEGIN_FAST, slow_body, 0)

    def fast_body(l, carry):
        plsc.subcore_barrier()
        row0 = (SMALLN - BEGIN_FAST * BIGN) + l * BIGN
        ws = pl.multiple_of((row0 - 72) * 2, 256)
        pltpu.sync_copy(table_hbm.at[pl.ds(ws + sid * FSEGW, FSEGW)],
                        sp_big.at[pl.ds(sid * FSEGW, FSEGW)])
        plsc.subcore_barrier()
        row0v = jnp.zeros((16,), jnp.int32) + row0
        wsv = jnp.zeros((16,), jnp.int32) + ws
        level_block(l, (bcast_f(l), row0v, wsv), True)
        return carry

    lax.fori_loop(BEGIN_FAST, NLEV, fast_body, 0)


_mesh = plsc.VectorSubcoreMesh(core_axis_name="c", subcore_axis_name="s")

_hash_kernel = functools.partial(
    pl.kernel,
    out_type=jax.ShapeDtypeStruct((NPTS * NLEV * NFEAT,), jnp.float32),
    mesh=_mesh,
    compiler_params=pltpu.CompilerParams(
        needs_layout_passes=False, use_tc_tiling_on_sc=False),
    scratch_types=[
        pltpu.VMEM((4 * C,), jnp.float32),
        pltpu.VMEM((4 * C,), jnp.float32),
        pltpu.VMEM((2 * NCORN * C,), jnp.int32),
        pltpu.VMEM((2 * NCORN * C,), jnp.int32),
        pltpu.VMEM((2 * NCORN * C,), jnp.float32),
        pltpu.VMEM((2 * NCORN * C,), jnp.float32),
        pltpu.VMEM((NCORN * C,), jnp.float32),
        pltpu.VMEM((NCORN * C,), jnp.float32),
        pltpu.VMEM((NFEAT * C,), jnp.float32),
        pltpu.VMEM((NFEAT * C,), jnp.float32),
        pltpu.VMEM((16,), jnp.float32),
        pltpu.VMEM((4 * 16,), jnp.int32),
        pltpu.VMEM_SHARED((BIGWP,), jnp.float32),
        pltpu.SemaphoreType.DMA,
        pltpu.SemaphoreType.DMA,
        pltpu.SemaphoreType.DMA,
        pltpu.SemaphoreType.DMA,
        pltpu.SemaphoreType.DMA,
        pltpu.SemaphoreType.DMA,
    ],
)(_body)


@jax.jit
def kernel(xyzs, table):
    xyz_p = jnp.pad(xyzs.reshape(NPTS // 128, 128, 3), ((0, 0), (0, 0), (0, 1)))
    xyz_c = jnp.transpose(xyz_p, (0, 2, 1)).reshape(-1)
    tab_p = jnp.pad(table, ((0, TOTP - TOT), (0, 0)))
    tab_c = jnp.transpose(tab_p.reshape(TOTP // 128, 128, 2), (0, 2, 1))
    tab_c = tab_c.reshape(-1)
    scales = jnp.asarray(np.array(SCALES, dtype=np.float32))
    sconst = np.zeros((4, 16), dtype=np.int32)
    for l in range(BEGIN_FAST):
        sconst[0, l] = RESS[l]
        sconst[1, l] = RESS[l] * RESS[l]
        sconst[2, l] = SZS[l]
        sconst[3, l] = OFFS[l]
    sconst = jnp.asarray(sconst.reshape(-1))
    out = _hash_kernel(xyz_c, tab_c, scales, sconst)
    x4 = out.reshape(NLEV, NPTS // 128, NFEAT, 128)
    return jnp.transpose(x4, (1, 3, 0, 2)).reshape(NPTS, NLEV, NFEAT)
